# baseline (speedup 1.0000x reference)
import functools

import jax
import jax.numpy as jnp
from jax import lax
from jax.experimental import pallas as pl
from jax.experimental.pallas import tpu as pltpu
from jax.experimental.pallas import tpu_sc as plsc

_N = 8192
_D = 256
_K = 8192
_BETA = 0.25
_BN = 1024
_BK = 2048
_L = 128


def _dist_argmin_body(z2_ref, e_ref, idx_ref, loss_ref,
                      zns, ens, runmin, runarg, acc,
                      *, nb, kb, bn, bk, ktot, scale):
    n = pl.program_id(0)
    k = pl.program_id(1)
    z2 = z2_ref[...]
    e = e_ref[...]

    @pl.when(k == 0)
    def _():
        zn = 0.25 * jnp.sum(z2 * z2, axis=1, keepdims=True)
        zns[...] = zn + jnp.zeros((bn, _L), jnp.float32)

    @pl.when(n == 0)
    def _():
        ens[0, pl.ds(k * bk, bk)] = jnp.sum(e * e, axis=1)

    t2 = lax.dot_general(z2, e, (((1,), (1,)), ((), ())),
                         preferred_element_type=jnp.float32)

    ngrp = bk // _L
    rc = min(128, bn)
    lane = lax.broadcasted_iota(jnp.int32, (rc, _L), 1)
    for r in range(bn // rc):
        rsl = pl.ds(r * rc, rc)
        znr = zns[rsl, :]
        best_v = None
        best_g = jnp.zeros((rc, _L), jnp.int32)
        for g in range(ngrp):
            en_g = ens[0, pl.ds(k * bk + g * _L, _L)][None, :]
            dv = (znr + en_g) - t2[r * rc:(r + 1) * rc, g * _L:(g + 1) * _L]
            if g == 0:
                best_v = dv
            else:
                lt = dv < best_v
                best_v = jnp.where(lt, dv, best_v)
                best_g = jnp.where(lt, jnp.int32(g), best_g)
        cand = best_g * _L + lane + k * bk

        @pl.when(k == 0)
        def _():
            runmin[rsl, :] = best_v
            runarg[rsl, :] = cand

        @pl.when(k > 0)
        def _():
            lt2 = best_v < runmin[rsl, :]
            runarg[rsl, :] = jnp.where(lt2, cand, runarg[rsl, :])
            runmin[rsl, :] = jnp.where(lt2, best_v, runmin[rsl, :])

    @pl.when(jnp.logical_and(n == 0, k == 0))
    def _():
        acc[0] = 0.0

    @pl.when(k == kb - 1)
    def _():
        rm = runmin[...]
        m = jnp.min(rm, axis=1, keepdims=True)
        c2 = jnp.where(rm == m, runarg[...], jnp.int32(2 ** 30))
        idx_ref[0, 0, :] = jnp.min(c2, axis=1)
        acc[0] += jnp.sum(m)

    @pl.when(jnp.logical_and(n == nb - 1, k == kb - 1))
    def _():
        loss_ref[...] = jnp.full((1, 1), acc[0] * scale, jnp.float32)


def _distance_argmin(zf, ew, bn=_BN, bk=_BK, interpret=False):
    n, d_dim = zf.shape
    k_dim = ew.shape[0]
    nb, kb = n // bn, k_dim // bk
    scale = (1.0 + _BETA) / (n * d_dim)
    body = functools.partial(_dist_argmin_body, nb=nb, kb=kb, bn=bn, bk=bk,
                             ktot=k_dim, scale=scale)
    return pl.pallas_call(
        body,
        grid=(nb, kb),
        in_specs=[
            pl.BlockSpec((bn, d_dim), lambda i, j: (i, 0)),
            pl.BlockSpec((bk, d_dim), lambda i, j: (j, 0)),
        ],
        out_specs=[
            pl.BlockSpec((1, 1, bn), lambda i, j: (i, 0, 0)),
            pl.BlockSpec((1, 1), lambda i, j: (0, 0)),
        ],
        out_shape=[
            jax.ShapeDtypeStruct((nb, 1, bn), jnp.int32),
            jax.ShapeDtypeStruct((1, 1), jnp.float32),
        ],
        scratch_shapes=[
            pltpu.VMEM((bn, _L), jnp.float32),
            pltpu.VMEM((1, k_dim), jnp.float32),
            pltpu.VMEM((bn, _L), jnp.float32),
            pltpu.VMEM((bn, _L), jnp.int32),
            pltpu.SMEM((1,), jnp.float32),
        ],
        interpret=interpret,
    )(zf * 2.0, ew)


def _sc_gather(table, idx):
    n = idx.shape[0]
    d_dim = table.shape[1]
    nw = 32
    b_per_w = n // nw
    ch = 128
    nch = b_per_w // ch
    mesh = plsc.VectorSubcoreMesh(core_axis_name="c", subcore_axis_name="s")

    @functools.partial(
        pl.kernel,
        mesh=mesh,
        out_type=jax.ShapeDtypeStruct((n, d_dim), jnp.float32),
        scratch_types=[
            pltpu.VMEM((ch,), jnp.int32),
            pltpu.VMEM((ch, d_dim), jnp.float32),
            pltpu.SemaphoreType.DMA,
        ],
    )
    def gather_kernel(table_hbm, idx_hbm, out_hbm, idx_v, rows_v, sem):
        wid = lax.axis_index("s") * 2 + lax.axis_index("c")
        base = wid * b_per_w
        for j in range(nch):
            off = base + j * ch
            pltpu.sync_copy(idx_hbm.at[pl.ds(off, ch)], idx_v)
            pltpu.async_copy(table_hbm.at[idx_v], rows_v, sem).wait()
            pltpu.sync_copy(rows_v, out_hbm.at[pl.ds(off, ch)])

    return gather_kernel(table, idx)


def kernel(z, embedding_weight):
    b, c, h, w = z.shape
    zp = jnp.transpose(z, (0, 2, 3, 1))
    zf = zp.reshape(-1, c)
    idx3, losssum = _distance_argmin(zf, embedding_weight)
    idx = idx3.reshape(-1)
    zq = _sc_gather(embedding_weight, idx)
    z_q_out = jnp.transpose(zq.reshape(b, h, w, c), (0, 3, 1, 2))
    loss = losssum[0, 0]
    return z_q_out, loss, idx


# speedup vs baseline: 1.2304x; 1.2304x over previous
import functools

import jax
import jax.numpy as jnp
from jax import lax
from jax.experimental import pallas as pl
from jax.experimental.pallas import tpu as pltpu
from jax.experimental.pallas import tpu_sc as plsc

_N = 8192
_D = 256
_K = 8192
_BETA = 0.25
_BN = 1024
_BK = 2048
_L = 128


def _dist_argmin_body(z2_ref, e_ref, idx_ref, loss_ref,
                      zns, ens, runmin, runarg, acc,
                      *, nb, kb, bn, bk, ktot, scale):
    n = pl.program_id(0)
    k = pl.program_id(1)
    z2 = z2_ref[...]
    e = e_ref[...]

    @pl.when(k == 0)
    def _():
        zn = 0.25 * jnp.sum(z2 * z2, axis=1, keepdims=True)
        zns[...] = zn + jnp.zeros((bn, _L), jnp.float32)

    @pl.when(n == 0)
    def _():
        ens[0, pl.ds(k * bk, bk)] = jnp.sum(e * e, axis=1)

    t2 = lax.dot_general(z2, e, (((1,), (1,)), ((), ())),
                         preferred_element_type=jnp.float32)

    ngrp = bk // _L
    rc = min(128, bn)
    lane = lax.broadcasted_iota(jnp.int32, (rc, _L), 1)
    for r in range(bn // rc):
        rsl = pl.ds(r * rc, rc)
        znr = zns[rsl, :]
        best_v = None
        best_g = jnp.zeros((rc, _L), jnp.int32)
        for g in range(ngrp):
            en_g = ens[0, pl.ds(k * bk + g * _L, _L)][None, :]
            dv = (znr + en_g) - t2[r * rc:(r + 1) * rc, g * _L:(g + 1) * _L]
            if g == 0:
                best_v = dv
            else:
                lt = dv < best_v
                best_v = jnp.where(lt, dv, best_v)
                best_g = jnp.where(lt, jnp.int32(g), best_g)
        cand = best_g * _L + lane + k * bk
        lt2 = jnp.logical_or(best_v < runmin[rsl, :], k == 0)
        runarg[rsl, :] = jnp.where(lt2, cand, runarg[rsl, :])
        runmin[rsl, :] = jnp.where(lt2, best_v, runmin[rsl, :])

    @pl.when(jnp.logical_and(n == 0, k == 0))
    def _():
        acc[0] = 0.0

    @pl.when(k == kb - 1)
    def _():
        rm = runmin[...]
        m = jnp.min(rm, axis=1, keepdims=True)
        c2 = jnp.where(rm == m, runarg[...], jnp.int32(2 ** 30))
        idx_ref[0, 0, :] = jnp.min(c2, axis=1)
        acc[0] += jnp.sum(m)

    @pl.when(jnp.logical_and(n == nb - 1, k == kb - 1))
    def _():
        loss_ref[...] = jnp.full((1, 1), acc[0] * scale, jnp.float32)


def _distance_argmin(zf, ew, bn=_BN, bk=_BK, interpret=False):
    n, d_dim = zf.shape
    k_dim = ew.shape[0]
    nb, kb = n // bn, k_dim // bk
    scale = (1.0 + _BETA) / (n * d_dim)
    body = functools.partial(_dist_argmin_body, nb=nb, kb=kb, bn=bn, bk=bk,
                             ktot=k_dim, scale=scale)
    return pl.pallas_call(
        body,
        grid=(nb, kb),
        in_specs=[
            pl.BlockSpec((bn, d_dim), lambda i, j: (i, 0)),
            pl.BlockSpec((bk, d_dim), lambda i, j: (j, 0)),
        ],
        out_specs=[
            pl.BlockSpec((1, 1, bn), lambda i, j: (i, 0, 0)),
            pl.BlockSpec((1, 1), lambda i, j: (0, 0)),
        ],
        out_shape=[
            jax.ShapeDtypeStruct((nb, 1, bn), jnp.int32),
            jax.ShapeDtypeStruct((1, 1), jnp.float32),
        ],
        scratch_shapes=[
            pltpu.VMEM((bn, _L), jnp.float32),
            pltpu.VMEM((1, k_dim), jnp.float32),
            pltpu.VMEM((bn, _L), jnp.float32),
            pltpu.VMEM((bn, _L), jnp.int32),
            pltpu.SMEM((1,), jnp.float32),
        ],
        interpret=interpret,
    )(zf * 2.0, ew)


def _sc_gather(table, idx):
    n = idx.shape[0]
    d_dim = table.shape[1]
    nw = 32
    b_per_w = n // nw
    ch = 128
    nch = b_per_w // ch
    mesh = plsc.VectorSubcoreMesh(core_axis_name="c", subcore_axis_name="s")

    @functools.partial(
        pl.kernel,
        mesh=mesh,
        out_type=jax.ShapeDtypeStruct((n, d_dim), jnp.float32),
        scratch_types=[
            pltpu.VMEM((ch,), jnp.int32),
            pltpu.VMEM((ch, d_dim), jnp.float32),
            pltpu.SemaphoreType.DMA,
        ],
    )
    def gather_kernel(table_hbm, idx_hbm, out_hbm, idx_v, rows_v, sem):
        wid = lax.axis_index("s") * 2 + lax.axis_index("c")
        base = wid * b_per_w
        for j in range(nch):
            off = base + j * ch
            pltpu.sync_copy(idx_hbm.at[pl.ds(off, ch)], idx_v)
            pltpu.async_copy(table_hbm.at[idx_v], rows_v, sem).wait()
            pltpu.sync_copy(rows_v, out_hbm.at[pl.ds(off, ch)])

    return gather_kernel(table, idx)


def kernel(z, embedding_weight):
    b, c, h, w = z.shape
    zp = jnp.transpose(z, (0, 2, 3, 1))
    zf = zp.reshape(-1, c)
    idx3, losssum = _distance_argmin(zf, embedding_weight)
    idx = idx3.reshape(-1)
    zq = _sc_gather(embedding_weight, idx)
    z_q_out = jnp.transpose(zq.reshape(b, h, w, c), (0, 3, 1, 2))
    loss = losssum[0, 0]
    return z_q_out, loss, idx
